# Initial kernel scaffold; baseline (speedup 1.0000x reference)
import jax
import jax.numpy as jnp
from jax.experimental import pallas as pl


def kernel(x, applyUMAP):
    raise NotImplementedError("write your pallas kernel here")


# speedup vs baseline: 2.3472x; 2.3472x over previous
import jax
import jax.numpy as jnp
from jax import lax
from jax.experimental import pallas as pl

_NB_VEC = 8


def _rv_kernel(x_ref, vec_ref, sim_ref, pos_ref):
    X = x_ref[0]
    C, H, W = X.shape
    flat = (lax.broadcasted_iota(jnp.int32, (H, W), 0) * W
            + lax.broadcasted_iota(jnp.int32, (H, W), 1))
    selpos = jnp.zeros((H, W), jnp.float32)
    score = jnp.zeros((H, W), jnp.float32)
    for i in range(_NB_VEC):
        if i == 0:
            n = jnp.int32((H * W) // 2)
        else:
            m = jnp.max(score)
            n = jnp.min(jnp.where(score >= m, flat, jnp.int32(2 ** 30)))
        oh = (flat == n).astype(jnp.float32)
        selpos = selpos + oh
        raw = jnp.sum(X * oh[None, :, :], axis=(1, 2), keepdims=True)
        diff = X - raw
        d2 = jnp.sum(diff * diff, axis=0)
        sim = jnp.exp(jnp.sqrt(d2) * (-1.0 / 20.0))
        wsum = jnp.sum(X * sim[None, :, :], axis=(1, 2))
        vec_ref[0, i, :] = wsum * (1.0 / jnp.sum(sim))
        sim_ref[0, i, :, :] = sim
        score = (1.0 - sim) if i == 0 else (1.0 - sim) * score
    pos_ref[0, 0, :, :] = selpos


def kernel(x, applyUMAP):
    B, C, H, W = x.shape
    vecs, sims, selpos = pl.pallas_call(
        _rv_kernel,
        grid=(B,),
        in_specs=[pl.BlockSpec((1, C, H, W), lambda b: (b, 0, 0, 0))],
        out_specs=[
            pl.BlockSpec((1, _NB_VEC, C), lambda b: (b, 0, 0)),
            pl.BlockSpec((1, _NB_VEC, H, W), lambda b: (b, 0, 0, 0)),
            pl.BlockSpec((1, 1, H, W), lambda b: (b, 0, 0, 0)),
        ],
        out_shape=[
            jax.ShapeDtypeStruct((B, _NB_VEC, C), jnp.float32),
            jax.ShapeDtypeStruct((B, _NB_VEC, H, W), jnp.float32),
            jax.ShapeDtypeStruct((B, 1, H, W), jnp.float32),
        ],
    )(x)
    return vecs, sims, selpos


# speedup vs baseline: 3.0041x; 1.2798x over previous
import jax
import jax.numpy as jnp
from jax import lax
from jax.experimental import pallas as pl

_NB_VEC = 8


def _rv_kernel(x_ref, vec_ref, sim_ref, pos_ref):
    X = x_ref[0]
    C, H, W = X.shape
    flat = (lax.broadcasted_iota(jnp.int32, (H, W), 0) * W
            + lax.broadcasted_iota(jnp.int32, (H, W), 1))
    lane = lax.broadcasted_iota(jnp.int32, (1, 1, W), 2)
    xn2 = jnp.sum(X * X, axis=0)
    selpos = jnp.zeros((H, W), jnp.float32)
    score = jnp.zeros((H, W), jnp.float32)
    for i in range(_NB_VEC):
        if i == 0:
            n = jnp.int32((H * W) // 2)
        else:
            m = jnp.max(score)
            n = jnp.min(jnp.where(score >= m, flat, jnp.int32(2 ** 30)))
        io = n // W
        jo = n % W
        selpos = selpos + (flat == n).astype(jnp.float32)
        row = x_ref[0, :, pl.ds(io, 1), :]
        raw = jnp.sum(row * (lane == jo).astype(jnp.float32),
                      axis=2, keepdims=True)
        rn2 = jnp.sum(raw * raw)
        dots = jnp.sum(X * raw, axis=0)
        d2 = jnp.maximum(xn2 - 2.0 * dots + rn2, 0.0)
        sim = jnp.exp(jnp.sqrt(d2) * (-1.0 / 20.0))
        wsum = jnp.sum(X * sim[None, :, :], axis=(1, 2))
        vec_ref[0, i, :] = wsum * (1.0 / jnp.sum(sim))
        sim_ref[0, i, :, :] = sim
        score = (1.0 - sim) if i == 0 else (1.0 - sim) * score
    pos_ref[0, 0, :, :] = selpos


def kernel(x, applyUMAP):
    B, C, H, W = x.shape
    vecs, sims, selpos = pl.pallas_call(
        _rv_kernel,
        grid=(B,),
        in_specs=[pl.BlockSpec((1, C, H, W), lambda b: (b, 0, 0, 0))],
        out_specs=[
            pl.BlockSpec((1, _NB_VEC, C), lambda b: (b, 0, 0)),
            pl.BlockSpec((1, _NB_VEC, H, W), lambda b: (b, 0, 0, 0)),
            pl.BlockSpec((1, 1, H, W), lambda b: (b, 0, 0, 0)),
        ],
        out_shape=[
            jax.ShapeDtypeStruct((B, _NB_VEC, C), jnp.float32),
            jax.ShapeDtypeStruct((B, _NB_VEC, H, W), jnp.float32),
            jax.ShapeDtypeStruct((B, 1, H, W), jnp.float32),
        ],
    )(x)
    return vecs, sims, selpos


# speedup vs baseline: 3.0049x; 1.0003x over previous
import jax
import jax.numpy as jnp
from jax import lax
from jax.experimental import pallas as pl
from jax.experimental.pallas import tpu as pltpu

_NB_VEC = 8


def _rv_kernel(x_ref, vec_ref, sim_ref, pos_ref):
    X = x_ref[0]
    C, H, W = X.shape
    flat = (lax.broadcasted_iota(jnp.int32, (H, W), 0) * W
            + lax.broadcasted_iota(jnp.int32, (H, W), 1))
    lane = lax.broadcasted_iota(jnp.int32, (1, 1, W), 2)
    xn2 = jnp.sum(X * X, axis=0)
    selpos = jnp.zeros((H, W), jnp.float32)
    score = jnp.zeros((H, W), jnp.float32)
    for i in range(_NB_VEC):
        if i == 0:
            n = jnp.int32((H * W) // 2)
        else:
            m = jnp.max(score)
            n = jnp.min(jnp.where(score >= m, flat, jnp.int32(2 ** 30)))
        io = n // W
        jo = n % W
        selpos = selpos + (flat == n).astype(jnp.float32)
        row = x_ref[0, :, pl.ds(io, 1), :]
        raw = jnp.sum(row * (lane == jo).astype(jnp.float32),
                      axis=2, keepdims=True)
        rn2 = jnp.sum(raw * raw)
        dots = jnp.sum(X * raw, axis=0)
        d2 = jnp.maximum(xn2 - 2.0 * dots + rn2, 0.0)
        sim = jnp.exp(jnp.sqrt(d2) * (-1.0 / 20.0))
        wsum = jnp.sum(X * sim[None, :, :], axis=(1, 2))
        vec_ref[0, i, :] = wsum * (1.0 / jnp.sum(sim))
        sim_ref[0, i, :, :] = sim
        score = (1.0 - sim) if i == 0 else (1.0 - sim) * score
    pos_ref[0, 0, :, :] = selpos


def kernel(x, applyUMAP):
    B, C, H, W = x.shape
    vecs, sims, selpos = pl.pallas_call(
        _rv_kernel,
        grid=(B,),
        in_specs=[pl.BlockSpec((1, C, H, W), lambda b: (b, 0, 0, 0))],
        out_specs=[
            pl.BlockSpec((1, _NB_VEC, C), lambda b: (b, 0, 0)),
            pl.BlockSpec((1, _NB_VEC, H, W), lambda b: (b, 0, 0, 0)),
            pl.BlockSpec((1, 1, H, W), lambda b: (b, 0, 0, 0)),
        ],
        out_shape=[
            jax.ShapeDtypeStruct((B, _NB_VEC, C), jnp.float32),
            jax.ShapeDtypeStruct((B, _NB_VEC, H, W), jnp.float32),
            jax.ShapeDtypeStruct((B, 1, H, W), jnp.float32),
        ],
        compiler_params=pltpu.CompilerParams(
            dimension_semantics=("parallel",),
        ),
    )(x)
    return vecs, sims, selpos
